# baseline (speedup 1.0000x reference)
import jax
import jax.numpy as jnp
from jax.experimental import pallas as pl
from jax.experimental.pallas import tpu as pltpu


def _stats_kernel(x_ref, n_ref, nv_ref, iw_ref, ib_ref, sc_ref, sh_ref):
    xb = x_ref[0]
    nb = n_ref[0]
    npix = float(xb.shape[1] * xb.shape[2])
    sx = jnp.sum(xb, axis=(1, 2))
    sxx = jnp.sum(xb * xb, axis=(1, 2))
    sxn = jnp.sum(xb * nb[None, :, :], axis=(1, 2))
    sn = jnp.sum(nb)
    snn = jnp.sum(nb * nb)
    a = nv_ref[0, 0]
    mean = (sx + a * sn) / npix
    ex2 = (sxx + 2.0 * a * sxn + a * a * snn) / npix
    var = ex2 - mean * mean
    rstd = 1.0 / jnp.sqrt(var + 1e-5)
    scl = iw_ref[0, 0] * rstd
    sc_ref[0, 0, 0] = scl
    sh_ref[0, 0, 0] = ib_ref[0, 0] - mean * scl


def _select_kernel(seg_ref, code_ref, out_ref):
    s = seg_ref[0]
    P, NR = s.shape
    jio = jax.lax.broadcasted_iota(jnp.int32, (P, NR), 1)
    jidx = jnp.max(jnp.where(s != 0, jio, -1), axis=1)
    oh = (jio == jidx[:, None]).astype(jnp.float32)
    out_ref[0] = jnp.dot(oh, code_ref[0], preferred_element_type=jnp.float32)


def _make_main_kernel(TH, H, W, C, SD, NH):
    HR = TH + 2

    def main_kernel(sp_ref, sc_ref, sn_ref, mt_ref, mc_ref, mb_ref,
                    x_ref, n_ref, nv_ref, scl_ref, shf_ref,
                    ws_ref, wg_ref, wm_ref, bm_ref, bgb_ref, out_ref,
                    sxt_ref, hx_ref, acc_ref, scol_ref, hcol_ref):
        i = pl.program_id(1)
        nt = pl.num_programs(1)
        f32 = jnp.float32

        bf16 = jnp.bfloat16
        sxt_ref[:, 0:1, :] = jnp.zeros((HR, 1, SD), bf16)
        sxt_ref[:, W + 1:W + 2, :] = jnp.zeros((HR, 1, SD), bf16)
        sxt_ref[0:1, 1:W + 1, :] = jnp.where(i > 0, sp_ref[0],
                                             jnp.zeros_like(sp_ref[0])).astype(bf16)
        sxt_ref[1:TH + 1, 1:W + 1, :] = sc_ref[0].astype(bf16)
        sxt_ref[TH + 1:HR, 1:W + 1, :] = jnp.where(i < nt - 1, sn_ref[0],
                                                   jnp.zeros_like(sn_ref[0])).astype(bf16)

        mtop = jnp.where(i > 0, mt_ref[0], jnp.zeros_like(mt_ref[0]))
        mbot = jnp.where(i < nt - 1, mb_ref[0], jnp.zeros_like(mb_ref[0]))
        mx = jnp.concatenate([mtop, mc_ref[0], mbot], axis=0)
        zm = jnp.zeros((TH + 4, 1, 3), f32)
        mx = jnp.concatenate([zm, mx, zm], axis=1).astype(bf16)

        mcol = jnp.concatenate(
            [mx[dy:dy + HR, dx:dx + W, :] for dy in range(3) for dx in range(3)],
            axis=2).reshape(HR * W, 27)
        hacc = jnp.dot(mcol, wm_ref[...], preferred_element_type=f32)
        hacc = hacc + bm_ref[...]
        hid = jnp.where(hacc >= 0, hacc, 0.2 * hacc).reshape(HR, W, NH)
        rio = jax.lax.broadcasted_iota(jnp.int32, (HR, 1, 1), 0)
        bad = ((rio == 0) & (i == 0)) | ((rio == HR - 1) & (i == nt - 1))
        hx_ref[:, 0:1, :] = jnp.zeros((HR, 1, NH), bf16)
        hx_ref[:, W + 1:W + 2, :] = jnp.zeros((HR, 1, NH), bf16)
        hx_ref[:, 1:W + 1, :] = jnp.where(bad, 0.0, hid).astype(bf16)

        for dy in range(3):
            for dx in range(3):
                t = dy * 3 + dx
                scol_ref[:, t * SD:(t + 1) * SD] = (
                    sxt_ref[dy:dy + TH, dx:dx + W, :].reshape(TH * W, SD))
                hcol_ref[:, t * NH:(t + 1) * NH] = (
                    hx_ref[dy:dy + TH, dx:dx + W, :].reshape(TH * W, NH))
        acc_ref[...] = jnp.dot(scol_ref[...], ws_ref[...],
                               preferred_element_type=f32)
        acc_ref[...] += jnp.dot(hcol_ref[...], wg_ref[...],
                                preferred_element_type=f32)

        acc = acc_ref[...] + bgb_ref[...]
        g3 = acc[:, :C].reshape(TH, W, C)
        b3 = acc[:, C:].reshape(TH, W, C)
        nvv = nv_ref[...][0]
        scl = scl_ref[0, 0]
        shf = shf_ref[0, 0]
        xt3 = jnp.transpose(x_ref[0], (1, 2, 0))
        xn3 = xt3 + n_ref[0][:, :, None] * nvv[None, None, :]
        xnorm = xn3 * scl[None, None, :] + shf[None, None, :]
        res = xnorm * (1.0 + g3) + b3
        out_ref[0] = jnp.transpose(res, (2, 0, 1))

    return main_kernel


def kernel(x, segmap_attentions, codes_vector, mask, noise, noise_var,
           in_w, in_b, ms_w, ms_b, mg_w, mg_b, mb_w, mb_b,
           cg_w, cg_b, cb_w, cb_b, blend_g, blend_b):
    B, C, H, W = x.shape
    NR = segmap_attentions.shape[1]
    SD = codes_vector.shape[2]
    NH = ms_w.shape[0]
    Hs = segmap_attentions.shape[2]
    Ws = segmap_attentions.shape[3]
    P = Hs * Ws
    f32 = jnp.float32

    TH = 16
    NT = H // TH
    TC = 32
    NC = C // TC

    ga = jax.nn.sigmoid(blend_g[0])
    ba = jax.nn.sigmoid(blend_b[0])

    cgT = jnp.transpose(cg_w, (2, 3, 1, 0)).reshape(9, SD, C)
    cbT = jnp.transpose(cb_w, (2, 3, 1, 0)).reshape(9, SD, C)
    w_style = jnp.concatenate([ga * cgT, ba * cbT],
                              axis=2).astype(jnp.bfloat16).reshape(9 * SD, 2 * C)
    mgT = jnp.transpose(mg_w, (2, 3, 1, 0)).reshape(9, NH, C)
    mbT = jnp.transpose(mb_w, (2, 3, 1, 0)).reshape(9, NH, C)
    w_gated = jnp.concatenate(
        [(1 - ga) * mgT, (1 - ba) * mbT],
        axis=2).astype(jnp.bfloat16).reshape(9 * NH, 2 * C)
    w_ms = jnp.transpose(ms_w, (2, 3, 1, 0)).reshape(27, NH).astype(jnp.bfloat16)
    b_gb = jnp.concatenate([ga * cg_b + (1 - ga) * mg_b,
                            ba * cb_b + (1 - ba) * mb_b])[None, :]
    b_ms = ms_b[None, :]

    noiseT = jnp.transpose(noise[:, :, :, 0], (0, 2, 1))
    seg_flat = jnp.transpose(segmap_attentions, (0, 2, 3, 1)).reshape(B, P, NR)
    mask_up = jnp.repeat(jnp.repeat(jnp.transpose(mask, (0, 2, 3, 1)),
                                    2, axis=1), 2, axis=2)
    nv2 = noise_var[None, :]

    nv3 = noise_var.reshape(NC, 1, TC)
    iw3 = in_w.reshape(NC, 1, TC)
    ib3 = in_b.reshape(NC, 1, TC)
    scale4, shift4 = pl.pallas_call(
        _stats_kernel,
        grid=(B, NC),
        in_specs=[
            pl.BlockSpec((1, TC, H, W), lambda b, c: (b, c, 0, 0)),
            pl.BlockSpec((1, H, W), lambda b, c: (b, 0, 0)),
            pl.BlockSpec((1, 1, TC), lambda b, c: (c, 0, 0)),
            pl.BlockSpec((1, 1, TC), lambda b, c: (c, 0, 0)),
            pl.BlockSpec((1, 1, TC), lambda b, c: (c, 0, 0)),
        ],
        out_specs=[
            pl.BlockSpec((1, 1, 1, TC), lambda b, c: (b, c, 0, 0)),
            pl.BlockSpec((1, 1, 1, TC), lambda b, c: (b, c, 0, 0)),
        ],
        out_shape=[
            jax.ShapeDtypeStruct((B, NC, 1, TC), f32),
            jax.ShapeDtypeStruct((B, NC, 1, TC), f32),
        ],
    )(x, noiseT, nv3, iw3, ib3)
    scale3 = scale4.reshape(B, 1, C)
    shift3 = shift4.reshape(B, 1, C)

    style_small = pl.pallas_call(
        _select_kernel,
        grid=(B,),
        in_specs=[
            pl.BlockSpec((1, P, NR), lambda b: (b, 0, 0)),
            pl.BlockSpec((1, NR, SD), lambda b: (b, 0, 0)),
        ],
        out_specs=pl.BlockSpec((1, P, SD), lambda b: (b, 0, 0)),
        out_shape=jax.ShapeDtypeStruct((B, P, SD), f32),
    )(seg_flat, codes_vector)

    style_up = style_small.reshape(B, Hs, Ws, SD)
    style_up = jnp.repeat(jnp.repeat(style_up, 2, axis=1), 2, axis=2)

    TH2 = TH // 2
    out_nhwc = pl.pallas_call(
        _make_main_kernel(TH, H, W, C, SD, NH),
        grid=(B, NT),
        in_specs=[
            pl.BlockSpec((1, 1, W, SD),
                         lambda b, i: (b, jnp.maximum(i * TH - 1, 0), 0, 0)),
            pl.BlockSpec((1, TH, W, SD), lambda b, i: (b, i, 0, 0)),
            pl.BlockSpec((1, 1, W, SD),
                         lambda b, i: (b, jnp.minimum((i + 1) * TH, H - 1), 0, 0)),
            pl.BlockSpec((1, 2, W, 3),
                         lambda b, i: (b, jnp.maximum(i * TH2 - 1, 0), 0, 0)),
            pl.BlockSpec((1, TH, W, 3), lambda b, i: (b, i, 0, 0)),
            pl.BlockSpec((1, 2, W, 3),
                         lambda b, i: (b, jnp.minimum((i + 1) * TH2, H // 2 - 1), 0, 0)),
            pl.BlockSpec((1, C, TH, W), lambda b, i: (b, 0, i, 0)),
            pl.BlockSpec((1, TH, W), lambda b, i: (b, i, 0)),
            pl.BlockSpec((1, C), lambda b, i: (0, 0)),
            pl.BlockSpec((1, 1, C), lambda b, i: (b, 0, 0)),
            pl.BlockSpec((1, 1, C), lambda b, i: (b, 0, 0)),
            pl.BlockSpec((9 * SD, 2 * C), lambda b, i: (0, 0)),
            pl.BlockSpec((9 * NH, 2 * C), lambda b, i: (0, 0)),
            pl.BlockSpec((27, NH), lambda b, i: (0, 0)),
            pl.BlockSpec((1, NH), lambda b, i: (0, 0)),
            pl.BlockSpec((1, 2 * C), lambda b, i: (0, 0)),
        ],
        out_specs=pl.BlockSpec((1, C, TH, W), lambda b, i: (b, 0, i, 0)),
        out_shape=jax.ShapeDtypeStruct((B, C, H, W), f32),
        scratch_shapes=[
            pltpu.VMEM((TH + 2, W + 2, SD), jnp.bfloat16),
            pltpu.VMEM((TH + 2, W + 2, NH), jnp.bfloat16),
            pltpu.VMEM((TH * W, 2 * C), f32),
            pltpu.VMEM((TH * W, 9 * SD), jnp.bfloat16),
            pltpu.VMEM((TH * W, 9 * NH), jnp.bfloat16),
        ],
    )(style_up, style_up, style_up, mask_up, mask_up, mask_up,
      x, noiseT, nv2, scale3, shift3,
      w_style, w_gated, w_ms, b_ms, b_gb)

    return out_nhwc


# speedup vs baseline: 1.4642x; 1.4642x over previous
import jax
import jax.numpy as jnp
from jax.experimental import pallas as pl
from jax.experimental.pallas import tpu as pltpu


def _stats_kernel(x_ref, n_ref, nv_ref, iw_ref, ib_ref, sc_ref, sh_ref):
    xb = x_ref[0]
    nb = n_ref[0]
    npix = float(xb.shape[1] * xb.shape[2])
    sx = jnp.sum(xb, axis=(1, 2))
    sxx = jnp.sum(xb * xb, axis=(1, 2))
    sxn = jnp.sum(xb * nb[None, :, :], axis=(1, 2))
    sn = jnp.sum(nb)
    snn = jnp.sum(nb * nb)
    a = nv_ref[0, 0]
    mean = (sx + a * sn) / npix
    ex2 = (sxx + 2.0 * a * sxn + a * a * snn) / npix
    var = ex2 - mean * mean
    rstd = 1.0 / jnp.sqrt(var + 1e-5)
    scl = iw_ref[0, 0] * rstd
    sc_ref[0, 0, 0] = scl
    sh_ref[0, 0, 0] = ib_ref[0, 0] - mean * scl


def _select_kernel(seg_ref, code_ref, out_ref):
    s = seg_ref[0]
    P, NR = s.shape
    jio = jax.lax.broadcasted_iota(jnp.int32, (P, NR), 1)
    jidx = jnp.max(jnp.where(s != 0, jio, -1), axis=1)
    oh = (jio == jidx[:, None]).astype(jnp.float32)
    out_ref[0] = jnp.dot(oh, code_ref[0],
                         preferred_element_type=jnp.float32).astype(out_ref.dtype)


def _make_main_kernel(TH, H, W, C, SD, NH):
    HR = TH + 2

    def main_kernel(sp_ref, sc_ref, sn_ref, mt_ref, mc_ref, mb_ref,
                    x_ref, n_ref, nv_ref, scl_ref, shf_ref,
                    ws_ref, wg_ref, wm_ref, bm_ref, bgb_ref, out_ref,
                    sxt_ref, hx_ref, acc_ref):
        i = pl.program_id(1)
        nt = pl.num_programs(1)
        f32 = jnp.float32

        bf16 = jnp.bfloat16
        sxt_ref[:, 0:1, :] = jnp.zeros((HR, 1, SD), bf16)
        sxt_ref[:, W + 1:W + 2, :] = jnp.zeros((HR, 1, SD), bf16)
        sxt_ref[0:1, 1:W + 1, :] = jnp.where(i > 0, sp_ref[0],
                                             jnp.zeros_like(sp_ref[0]))
        sxt_ref[1:TH + 1, 1:W + 1, :] = sc_ref[0]
        sxt_ref[TH + 1:HR, 1:W + 1, :] = jnp.where(i < nt - 1, sn_ref[0],
                                                   jnp.zeros_like(sn_ref[0]))

        mtop = jnp.where(i > 0, mt_ref[0], jnp.zeros_like(mt_ref[0]))
        mbot = jnp.where(i < nt - 1, mb_ref[0], jnp.zeros_like(mb_ref[0]))
        mx = jnp.concatenate([mtop, mc_ref[0], mbot], axis=0)
        zm = jnp.zeros((TH + 4, 1, 3), bf16)
        mx = jnp.concatenate([zm, mx, zm], axis=1)

        mcol = jnp.concatenate(
            [mx[dy:dy + HR, dx:dx + W, :] for dy in range(3) for dx in range(3)],
            axis=2).reshape(HR * W, 27)
        hacc = jnp.dot(mcol, wm_ref[...], preferred_element_type=f32)
        hacc = hacc + bm_ref[...]
        hid = jnp.where(hacc >= 0, hacc, 0.2 * hacc).reshape(HR, W, NH)
        rio = jax.lax.broadcasted_iota(jnp.int32, (HR, 1, 1), 0)
        bad = ((rio == 0) & (i == 0)) | ((rio == HR - 1) & (i == nt - 1))
        hx_ref[:, 0:1, :] = jnp.zeros((HR, 1, NH), bf16)
        hx_ref[:, W + 1:W + 2, :] = jnp.zeros((HR, 1, NH), bf16)
        hx_ref[:, 1:W + 1, :] = jnp.where(bad, 0.0, hid).astype(bf16)

        acc_ref[...] = jnp.broadcast_to(bgb_ref[...], (TH * W, 2 * C))
        for dy in range(3):
            for dx in range(3):
                t = dy * 3 + dx
                ps = sxt_ref[dy:dy + TH, dx:dx + W, :].reshape(TH * W, SD)
                ph = hx_ref[dy:dy + TH, dx:dx + W, :].reshape(TH * W, NH)
                acc_ref[...] += (
                    jnp.dot(ps, ws_ref[t], preferred_element_type=f32)
                    + jnp.dot(ph, wg_ref[t], preferred_element_type=f32))

        acc = acc_ref[...]
        g3 = acc[:, :C].reshape(TH, W, C)
        b3 = acc[:, C:].reshape(TH, W, C)
        nvv = nv_ref[...][0]
        scl = scl_ref[0, 0]
        shf = shf_ref[0, 0]
        xt3 = jnp.transpose(x_ref[0], (1, 2, 0))
        xn3 = xt3 + n_ref[0][:, :, None] * nvv[None, None, :]
        xnorm = xn3 * scl[None, None, :] + shf[None, None, :]
        res = xnorm * (1.0 + g3) + b3
        out_ref[0] = jnp.transpose(res, (2, 0, 1))

    return main_kernel


def kernel(x, segmap_attentions, codes_vector, mask, noise, noise_var,
           in_w, in_b, ms_w, ms_b, mg_w, mg_b, mb_w, mb_b,
           cg_w, cg_b, cb_w, cb_b, blend_g, blend_b):
    B, C, H, W = x.shape
    NR = segmap_attentions.shape[1]
    SD = codes_vector.shape[2]
    NH = ms_w.shape[0]
    Hs = segmap_attentions.shape[2]
    Ws = segmap_attentions.shape[3]
    P = Hs * Ws
    f32 = jnp.float32

    TH = 32
    NT = H // TH
    TC = 32
    NC = C // TC

    ga = jax.nn.sigmoid(blend_g[0])
    ba = jax.nn.sigmoid(blend_b[0])

    cgT = jnp.transpose(cg_w, (2, 3, 1, 0)).reshape(9, SD, C)
    cbT = jnp.transpose(cb_w, (2, 3, 1, 0)).reshape(9, SD, C)
    w_style = jnp.concatenate([ga * cgT, ba * cbT], axis=2).astype(jnp.bfloat16)
    mgT = jnp.transpose(mg_w, (2, 3, 1, 0)).reshape(9, NH, C)
    mbT = jnp.transpose(mb_w, (2, 3, 1, 0)).reshape(9, NH, C)
    w_gated = jnp.concatenate([(1 - ga) * mgT,
                               (1 - ba) * mbT], axis=2).astype(jnp.bfloat16)
    w_ms = jnp.transpose(ms_w, (2, 3, 1, 0)).reshape(27, NH).astype(jnp.bfloat16)
    b_gb = jnp.concatenate([ga * cg_b + (1 - ga) * mg_b,
                            ba * cb_b + (1 - ba) * mb_b])[None, :]
    b_ms = ms_b[None, :]

    noiseT = jnp.transpose(noise[:, :, :, 0], (0, 2, 1))
    seg_flat = jnp.transpose(segmap_attentions, (0, 2, 3, 1)).reshape(B, P, NR)
    mask_up = jnp.repeat(jnp.repeat(jnp.transpose(mask, (0, 2, 3, 1)),
                                    2, axis=1), 2,
                         axis=2).astype(jnp.bfloat16)
    nv2 = noise_var[None, :]

    nv3 = noise_var.reshape(NC, 1, TC)
    iw3 = in_w.reshape(NC, 1, TC)
    ib3 = in_b.reshape(NC, 1, TC)
    scale4, shift4 = pl.pallas_call(
        _stats_kernel,
        grid=(B, NC),
        in_specs=[
            pl.BlockSpec((1, TC, H, W), lambda b, c: (b, c, 0, 0)),
            pl.BlockSpec((1, H, W), lambda b, c: (b, 0, 0)),
            pl.BlockSpec((1, 1, TC), lambda b, c: (c, 0, 0)),
            pl.BlockSpec((1, 1, TC), lambda b, c: (c, 0, 0)),
            pl.BlockSpec((1, 1, TC), lambda b, c: (c, 0, 0)),
        ],
        out_specs=[
            pl.BlockSpec((1, 1, 1, TC), lambda b, c: (b, c, 0, 0)),
            pl.BlockSpec((1, 1, 1, TC), lambda b, c: (b, c, 0, 0)),
        ],
        out_shape=[
            jax.ShapeDtypeStruct((B, NC, 1, TC), f32),
            jax.ShapeDtypeStruct((B, NC, 1, TC), f32),
        ],
    )(x, noiseT, nv3, iw3, ib3)
    scale3 = scale4.reshape(B, 1, C)
    shift3 = shift4.reshape(B, 1, C)

    style_small = pl.pallas_call(
        _select_kernel,
        grid=(B,),
        in_specs=[
            pl.BlockSpec((1, P, NR), lambda b: (b, 0, 0)),
            pl.BlockSpec((1, NR, SD), lambda b: (b, 0, 0)),
        ],
        out_specs=pl.BlockSpec((1, P, SD), lambda b: (b, 0, 0)),
        out_shape=jax.ShapeDtypeStruct((B, P, SD), jnp.bfloat16),
    )(seg_flat, codes_vector)

    style_up = style_small.reshape(B, Hs, Ws, SD)
    style_up = jnp.repeat(jnp.repeat(style_up, 2, axis=1), 2, axis=2)

    TH2 = TH // 2
    out_nhwc = pl.pallas_call(
        _make_main_kernel(TH, H, W, C, SD, NH),
        grid=(B, NT),
        in_specs=[
            pl.BlockSpec((1, 1, W, SD),
                         lambda b, i: (b, jnp.maximum(i * TH - 1, 0), 0, 0)),
            pl.BlockSpec((1, TH, W, SD), lambda b, i: (b, i, 0, 0)),
            pl.BlockSpec((1, 1, W, SD),
                         lambda b, i: (b, jnp.minimum((i + 1) * TH, H - 1), 0, 0)),
            pl.BlockSpec((1, 2, W, 3),
                         lambda b, i: (b, jnp.maximum(i * TH2 - 1, 0), 0, 0)),
            pl.BlockSpec((1, TH, W, 3), lambda b, i: (b, i, 0, 0)),
            pl.BlockSpec((1, 2, W, 3),
                         lambda b, i: (b, jnp.minimum((i + 1) * TH2, H // 2 - 1), 0, 0)),
            pl.BlockSpec((1, C, TH, W), lambda b, i: (b, 0, i, 0)),
            pl.BlockSpec((1, TH, W), lambda b, i: (b, i, 0)),
            pl.BlockSpec((1, C), lambda b, i: (0, 0)),
            pl.BlockSpec((1, 1, C), lambda b, i: (b, 0, 0)),
            pl.BlockSpec((1, 1, C), lambda b, i: (b, 0, 0)),
            pl.BlockSpec((9, SD, 2 * C), lambda b, i: (0, 0, 0)),
            pl.BlockSpec((9, NH, 2 * C), lambda b, i: (0, 0, 0)),
            pl.BlockSpec((27, NH), lambda b, i: (0, 0)),
            pl.BlockSpec((1, NH), lambda b, i: (0, 0)),
            pl.BlockSpec((1, 2 * C), lambda b, i: (0, 0)),
        ],
        out_specs=pl.BlockSpec((1, C, TH, W), lambda b, i: (b, 0, i, 0)),
        out_shape=jax.ShapeDtypeStruct((B, C, H, W), f32),
        scratch_shapes=[
            pltpu.VMEM((TH + 2, W + 2, SD), jnp.bfloat16),
            pltpu.VMEM((TH + 2, W + 2, NH), jnp.bfloat16),
            pltpu.VMEM((TH * W, 2 * C), f32),
        ],
    )(style_up, style_up, style_up, mask_up, mask_up, mask_up,
      x, noiseT, nv2, scale3, shift3,
      w_style, w_gated, w_ms, b_ms, b_gb)

    return out_nhwc
